# initial kernel scaffold (unmeasured)
import functools

import jax
import jax.numpy as jnp
from jax import lax
from jax.experimental import pallas as pl
from jax.experimental.pallas import tpu as pltpu

N_DEV = 8
E_LOC = 8
N_EXP = N_DEV * E_LOC
C = 128
D = 512
H = 1024


def _ring_moe(xg, w_loc):

    def body(xg_hbm, w_hbm, y_hbm, comm, xg_s, y_s,
             send_sems, recv_sems, load_sem, store_sem, credit_sem):
        my = lax.axis_index("i")
        left = lax.rem(my + (N_DEV - 1), N_DEV)
        right = lax.rem(my + 1, N_DEV)

        barrier = pltpu.get_barrier_semaphore()
        for nbr in (left, right):
            pl.semaphore_signal(barrier, inc=1, device_id=(nbr,),
                                device_id_type=pl.DeviceIdType.MESH)
        pl.semaphore_wait(barrier, 2)

        init = pltpu.make_async_copy(w_hbm, comm.at[0], load_sem)
        init.start()
        init.wait()

        for s in range(N_DEV):
            cur, nxt = s % 2, (s + 1) % 2
            src = lax.rem(my + (N_DEV - s), N_DEV)

            ld = pltpu.make_async_copy(
                xg_hbm.at[pl.ds(src * E_LOC, E_LOC)], xg_s, load_sem)
            ld.start()
            ld.wait()

            if s < N_DEV - 1:
                if s >= 1:
                    pl.semaphore_wait(credit_sem, 1)
                rdma = pltpu.make_async_remote_copy(
                    src_ref=comm.at[cur], dst_ref=comm.at[nxt],
                    send_sem=send_sems.at[cur], recv_sem=recv_sems.at[nxt],
                    device_id=(right,), device_id_type=pl.DeviceIdType.MESH)
                rdma.start()

            for j in range(E_LOC):
                y_s[j] = jnp.dot(xg_s[j], comm[cur, j],
                                 preferred_element_type=jnp.float32)

            st = pltpu.make_async_copy(
                y_s, y_hbm.at[pl.ds(src * E_LOC, E_LOC)], store_sem)
            st.start()
            st.wait()

            if s < N_DEV - 1:
                rdma.wait()
                if s < N_DEV - 2:
                    pl.semaphore_signal(credit_sem, inc=1, device_id=(left,),
                                        device_id_type=pl.DeviceIdType.MESH)

        @functools.partial(pl.run_scoped,
                           exit_sem=pltpu.SemaphoreType.REGULAR)
        def _(exit_sem):
            for nbr in (left, right):
                pl.semaphore_signal(exit_sem, inc=1, device_id=(nbr,),
                                    device_id_type=pl.DeviceIdType.MESH)
            pl.semaphore_wait(exit_sem, 2)

    return pl.pallas_call(
        body,
        out_shape=jax.ShapeDtypeStruct((N_EXP, C, H), jnp.float32),
        in_specs=[pl.BlockSpec(memory_space=pltpu.ANY),
                  pl.BlockSpec(memory_space=pltpu.ANY)],
        out_specs=pl.BlockSpec(memory_space=pltpu.ANY),
        scratch_shapes=[
            pltpu.VMEM((2, E_LOC, D, H), jnp.float32),
            pltpu.VMEM((E_LOC, C, D), jnp.float32),
            pltpu.VMEM((E_LOC, C, H), jnp.float32),
            pltpu.SemaphoreType.DMA((2,)),
            pltpu.SemaphoreType.DMA((2,)),
            pltpu.SemaphoreType.DMA,
            pltpu.SemaphoreType.DMA,
            pltpu.SemaphoreType.REGULAR,
        ],
        compiler_params=pltpu.CompilerParams(collective_id=0),
    )(xg, w_loc)


def kernel(x, router_W, route_idx, expert_W):
    n = x.shape[0]

    scores = x @ router_W
    probs = jax.nn.softmax(scores, axis=-1)
    g = jnp.take_along_axis(probs, route_idx, axis=1)
    w = g / jnp.sum(g, axis=-1, keepdims=True)

    eid = route_idx.reshape(-1)
    tok = jnp.arange(2 * n, dtype=jnp.int32) // 2
    wf = w.reshape(-1)

    order = jnp.argsort(eid)
    sorted_eid = eid[order]
    counts = jnp.bincount(eid, length=N_EXP)
    starts = jnp.cumsum(counts) - counts
    rank = jnp.arange(2 * n, dtype=jnp.int32) - starts[sorted_eid].astype(jnp.int32)

    pos_sorted = jnp.where(rank < C, sorted_eid * C + rank, N_EXP * C)
    pos = jnp.zeros(2 * n, dtype=jnp.int32).at[order].set(
        pos_sorted.astype(jnp.int32))

    rows = x[tok] * wf[:, None]
    xg_flat = jnp.zeros((N_EXP * C + 1, D), jnp.float32).at[pos].set(rows)
    xg = xg_flat[:N_EXP * C].reshape(N_EXP, C, D)

    y = _ring_moe(xg, expert_W)

    y_ext = jnp.concatenate(
        [y.reshape(N_EXP * C, H), jnp.zeros((1, H), jnp.float32)], axis=0)
    pos2 = pos.reshape(n, 2)
    return y_ext[pos2[:, 0]] + y_ext[pos2[:, 1]]


# baseline (device time: 1874512 ns/iter reference)
import functools

import jax
import jax.numpy as jnp
from jax import lax
from jax.experimental import pallas as pl
from jax.experimental.pallas import tpu as pltpu

N_DEV = 8
E_LOC = 8
N_EXP = N_DEV * E_LOC
C = 128
D = 512
H = 1024


def _ring_moe(xg, w_loc):

    def body(xg_hbm, w_hbm, y_hbm, comm, xg_s, y_s,
             send_sems, recv_sems, load_sem, store_sem, credit_sem):
        my = lax.axis_index("i")
        left = lax.rem(my + (N_DEV - 1), N_DEV)
        right = lax.rem(my + 1, N_DEV)

        barrier = pltpu.get_barrier_semaphore()
        for nbr in (left, right):
            pl.semaphore_signal(barrier, inc=1, device_id=(nbr,),
                                device_id_type=pl.DeviceIdType.MESH)
        pl.semaphore_wait(barrier, 2)

        init = pltpu.make_async_copy(w_hbm, comm.at[0], load_sem)
        init.start()
        init.wait()

        for s in range(N_DEV):
            cur, nxt = s % 2, (s + 1) % 2
            src = lax.rem(my + (N_DEV - s), N_DEV)

            ld = pltpu.make_async_copy(
                xg_hbm.at[pl.ds(src * E_LOC, E_LOC)], xg_s, load_sem)
            ld.start()
            ld.wait()

            if s < N_DEV - 1:
                if s >= 1:
                    pl.semaphore_wait(credit_sem, 1)
                rdma = pltpu.make_async_remote_copy(
                    src_ref=comm.at[cur], dst_ref=comm.at[nxt],
                    send_sem=send_sems.at[cur], recv_sem=recv_sems.at[nxt],
                    device_id=(right,), device_id_type=pl.DeviceIdType.MESH)
                rdma.start()

            for j in range(E_LOC):
                y_s[j] = jnp.dot(xg_s[j], comm[cur, j],
                                 preferred_element_type=jnp.float32)

            st = pltpu.make_async_copy(
                y_s, y_hbm.at[pl.ds(src * E_LOC, E_LOC)], store_sem)
            st.start()
            st.wait()

            if s < N_DEV - 1:
                rdma.wait()
                if s < N_DEV - 2:
                    pl.semaphore_signal(credit_sem, inc=1, device_id=(left,),
                                        device_id_type=pl.DeviceIdType.MESH)

        @functools.partial(pl.run_scoped,
                           exit_sem=pltpu.SemaphoreType.REGULAR)
        def _(exit_sem):
            for nbr in (left, right):
                pl.semaphore_signal(exit_sem, inc=1, device_id=(nbr,),
                                    device_id_type=pl.DeviceIdType.MESH)
            pl.semaphore_wait(exit_sem, 2)

    return pl.pallas_call(
        body,
        out_shape=jax.ShapeDtypeStruct((N_EXP, C, H), jnp.float32),
        in_specs=[pl.BlockSpec(memory_space=pl.ANY),
                  pl.BlockSpec(memory_space=pl.ANY)],
        out_specs=pl.BlockSpec(memory_space=pl.ANY),
        scratch_shapes=[
            pltpu.VMEM((2, E_LOC, D, H), jnp.float32),
            pltpu.VMEM((E_LOC, C, D), jnp.float32),
            pltpu.VMEM((E_LOC, C, H), jnp.float32),
            pltpu.SemaphoreType.DMA((2,)),
            pltpu.SemaphoreType.DMA((2,)),
            pltpu.SemaphoreType.DMA,
            pltpu.SemaphoreType.DMA,
            pltpu.SemaphoreType.REGULAR,
        ],
        compiler_params=pltpu.CompilerParams(
            collective_id=0, vmem_limit_bytes=60 * 1024 * 1024),
    )(xg, w_loc)


def kernel(x, router_W, route_idx, expert_W):
    n = x.shape[0]

    scores = x @ router_W
    probs = jax.nn.softmax(scores, axis=-1)
    g = jnp.take_along_axis(probs, route_idx, axis=1)
    w = g / jnp.sum(g, axis=-1, keepdims=True)

    eid = route_idx.reshape(-1)
    tok = jnp.arange(2 * n, dtype=jnp.int32) // 2
    wf = w.reshape(-1)

    order = jnp.argsort(eid)
    sorted_eid = eid[order]
    counts = jnp.bincount(eid, length=N_EXP)
    starts = jnp.cumsum(counts) - counts
    rank = jnp.arange(2 * n, dtype=jnp.int32) - starts[sorted_eid].astype(jnp.int32)

    pos_sorted = jnp.where(rank < C, sorted_eid * C + rank, N_EXP * C)
    pos = jnp.zeros(2 * n, dtype=jnp.int32).at[order].set(
        pos_sorted.astype(jnp.int32))

    rows = x[tok] * wf[:, None]
    xg_flat = jnp.zeros((N_EXP * C + 1, D), jnp.float32).at[pos].set(rows)
    xg = xg_flat[:N_EXP * C].reshape(N_EXP, C, D)

    y = _ring_moe(xg, expert_W)

    y_ext = jnp.concatenate(
        [y.reshape(N_EXP * C, H), jnp.zeros((1, H), jnp.float32)], axis=0)
    pos2 = pos.reshape(n, 2)
    return y_ext[pos2[:, 0]] + y_ext[pos2[:, 1]]


# device time: 1332013 ns/iter; 1.4073x vs baseline; 1.4073x over previous
import functools

import jax
import jax.numpy as jnp
from jax import lax
from jax.experimental import pallas as pl
from jax.experimental.pallas import tpu as pltpu

N_DEV = 8
E_LOC = 8
N_EXP = N_DEV * E_LOC
D = 512
H = 1024


def _ring_moe(x, wg, w_loc):
    n = x.shape[0]

    def body(x_ref, wg_ref, w_hbm, out_ref, comm,
             send_sems, recv_sems, load_sem, credit_sem):
        s = pl.program_id(0)
        my = lax.axis_index("i")
        left = lax.rem(my + (N_DEV - 1), N_DEV)
        right = lax.rem(my + 1, N_DEV)
        cur = lax.rem(s, 2)
        nxt = lax.rem(s + 1, 2)

        @pl.when(s == 0)
        def _():
            barrier = pltpu.get_barrier_semaphore()
            for nbr in (left, right):
                pl.semaphore_signal(barrier, inc=1, device_id=(nbr,),
                                    device_id_type=pl.DeviceIdType.MESH)
            pl.semaphore_wait(barrier, 2)
            init = pltpu.make_async_copy(w_hbm, comm.at[0], load_sem)
            init.start()
            init.wait()
            out_ref[:, :] = jnp.zeros_like(out_ref)

        rdma = pltpu.make_async_remote_copy(
            src_ref=comm.at[cur], dst_ref=comm.at[nxt],
            send_sem=send_sems.at[cur], recv_sem=recv_sems.at[nxt],
            device_id=(right,), device_id_type=pl.DeviceIdType.MESH)

        @pl.when(s < N_DEV - 1)
        def _():
            @pl.when(s >= 1)
            def _():
                pl.semaphore_wait(credit_sem, 1)
            rdma.start()

        xv = x_ref[:, :]
        acc = None
        for j in range(E_LOC):
            yj = jnp.dot(xv, comm[cur, j], preferred_element_type=jnp.float32)
            col = wg_ref[0, :, j:j + 1]
            acc = col * yj if acc is None else acc + col * yj
        out_ref[:, :] = out_ref[:, :] + acc

        @pl.when(s < N_DEV - 1)
        def _():
            rdma.wait()

            @pl.when(s < N_DEV - 2)
            def _():
                pl.semaphore_signal(credit_sem, inc=1, device_id=(left,),
                                    device_id_type=pl.DeviceIdType.MESH)

        @pl.when(s == N_DEV - 1)
        def _():
            @functools.partial(pl.run_scoped,
                               exit_sem=pltpu.SemaphoreType.REGULAR)
            def _(exit_sem):
                for nbr in (left, right):
                    pl.semaphore_signal(exit_sem, inc=1, device_id=(nbr,),
                                        device_id_type=pl.DeviceIdType.MESH)
                pl.semaphore_wait(exit_sem, 2)

    return pl.pallas_call(
        body,
        grid=(N_DEV,),
        out_shape=jax.ShapeDtypeStruct((n, H), jnp.float32),
        in_specs=[
            pl.BlockSpec((n, D), lambda s: (0, 0)),
            pl.BlockSpec((1, n, E_LOC), lambda s: (s, 0, 0)),
            pl.BlockSpec(memory_space=pl.ANY),
        ],
        out_specs=pl.BlockSpec((n, H), lambda s: (0, 0)),
        scratch_shapes=[
            pltpu.VMEM((2, E_LOC, D, H), jnp.float32),
            pltpu.SemaphoreType.DMA((2,)),
            pltpu.SemaphoreType.DMA((2,)),
            pltpu.SemaphoreType.DMA,
            pltpu.SemaphoreType.REGULAR,
        ],
        compiler_params=pltpu.CompilerParams(
            collective_id=0,
            vmem_limit_bytes=60 * 1024 * 1024,
            dimension_semantics=("arbitrary",),
        ),
    )(x, wg, w_loc)


def kernel(x, router_W, route_idx, expert_W):
    n = x.shape[0]

    scores = x @ router_W
    probs = jax.nn.softmax(scores, axis=-1)

    eids = jnp.arange(N_EXP, dtype=route_idx.dtype)[None, :]
    oh0 = route_idx[:, 0:1] == eids
    oh1 = route_idx[:, 1:2] == eids
    g0 = jnp.sum(jnp.where(oh0, probs, 0.0), axis=-1, keepdims=True)
    g1 = jnp.sum(jnp.where(oh1, probs, 0.0), axis=-1, keepdims=True)
    gs = g0 + g1
    w_dense = jnp.where(oh0, g0 / gs, 0.0) + jnp.where(oh1, g1 / gs, 0.0)

    my = lax.axis_index("i")
    wg = w_dense.reshape(n, N_DEV, E_LOC)[:, ::-1, :]
    wg = jnp.roll(wg, shift=my + 1, axis=1)
    wg = jnp.transpose(wg, (1, 0, 2))
    return _ring_moe(x, wg, expert_W)


# device time: 708890 ns/iter; 2.6443x vs baseline; 1.8790x over previous
import functools

import jax
import jax.numpy as jnp
from jax import lax
from jax.experimental import pallas as pl
from jax.experimental.pallas import tpu as pltpu

N_DEV = 8
E_LOC = 8
N_EXP = N_DEV * E_LOC
D = 512
H = 1024


def _ring_moe(x, wg, w_loc):
    n = x.shape[0]

    def body(x_ref, wg_ref, w_hbm, out_ref, comm,
             send_sems, recv_sems, load_sem, credit_sem):
        s = pl.program_id(0)
        my = lax.axis_index("i")
        left = lax.rem(my + (N_DEV - 1), N_DEV)
        right = lax.rem(my + 1, N_DEV)
        cur = lax.rem(s, 2)
        nxt = lax.rem(s + 1, 2)

        @pl.when(s == 0)
        def _():
            barrier = pltpu.get_barrier_semaphore()
            for nbr in (left, right):
                pl.semaphore_signal(barrier, inc=1, device_id=(nbr,),
                                    device_id_type=pl.DeviceIdType.MESH)
            pl.semaphore_wait(barrier, 2)
            init = pltpu.make_async_copy(w_hbm, comm.at[0], load_sem)
            init.start()
            init.wait()
            out_ref[:, :] = jnp.zeros_like(out_ref)

        rdma = pltpu.make_async_remote_copy(
            src_ref=comm.at[cur], dst_ref=comm.at[nxt],
            send_sem=send_sems.at[cur], recv_sem=recv_sems.at[nxt],
            device_id=(right,), device_id_type=pl.DeviceIdType.MESH)

        @pl.when(s < N_DEV - 1)
        def _():
            @pl.when(s >= 1)
            def _():
                pl.semaphore_wait(credit_sem, 1)
            rdma.start()

        xv = x_ref[:, :]
        acc = None
        for j in range(E_LOC):
            yj = jnp.dot(xv, comm[cur, j], preferred_element_type=jnp.float32)
            yj = yj.astype(jnp.float32)
            col = wg_ref[0, :, j:j + 1]
            acc = col * yj if acc is None else acc + col * yj
        out_ref[:, :] = out_ref[:, :] + acc

        @pl.when(s < N_DEV - 1)
        def _():
            rdma.wait()

            @pl.when(s < N_DEV - 2)
            def _():
                pl.semaphore_signal(credit_sem, inc=1, device_id=(left,),
                                    device_id_type=pl.DeviceIdType.MESH)

        @pl.when(s == N_DEV - 1)
        def _():
            @functools.partial(pl.run_scoped,
                               exit_sem=pltpu.SemaphoreType.REGULAR)
            def _(exit_sem):
                for nbr in (left, right):
                    pl.semaphore_signal(exit_sem, inc=1, device_id=(nbr,),
                                        device_id_type=pl.DeviceIdType.MESH)
                pl.semaphore_wait(exit_sem, 2)

    return pl.pallas_call(
        body,
        grid=(N_DEV,),
        out_shape=jax.ShapeDtypeStruct((n, H), jnp.float32),
        in_specs=[
            pl.BlockSpec((n, D), lambda s: (0, 0)),
            pl.BlockSpec((1, n, E_LOC), lambda s: (s, 0, 0)),
            pl.BlockSpec(memory_space=pl.ANY),
        ],
        out_specs=pl.BlockSpec((n, H), lambda s: (0, 0)),
        scratch_shapes=[
            pltpu.VMEM((2, E_LOC, D, H), jnp.bfloat16),
            pltpu.SemaphoreType.DMA((2,)),
            pltpu.SemaphoreType.DMA((2,)),
            pltpu.SemaphoreType.DMA,
            pltpu.SemaphoreType.REGULAR,
        ],
        compiler_params=pltpu.CompilerParams(
            collective_id=0,
            vmem_limit_bytes=60 * 1024 * 1024,
            dimension_semantics=("arbitrary",),
        ),
    )(x, wg, w_loc)


def kernel(x, router_W, route_idx, expert_W):
    n = x.shape[0]

    scores = x @ router_W
    probs = jax.nn.softmax(scores, axis=-1)

    eids = jnp.arange(N_EXP, dtype=route_idx.dtype)[None, :]
    oh0 = route_idx[:, 0:1] == eids
    oh1 = route_idx[:, 1:2] == eids
    g0 = jnp.sum(jnp.where(oh0, probs, 0.0), axis=-1, keepdims=True)
    g1 = jnp.sum(jnp.where(oh1, probs, 0.0), axis=-1, keepdims=True)
    gs = g0 + g1
    w_dense = jnp.where(oh0, g0 / gs, 0.0) + jnp.where(oh1, g1 / gs, 0.0)

    my = lax.axis_index("i")
    wg = w_dense.reshape(n, N_DEV, E_LOC)[:, ::-1, :]
    wg = jnp.roll(wg, shift=my + 1, axis=1)
    wg = jnp.transpose(wg, (1, 0, 2))
    return _ring_moe(x.astype(jnp.bfloat16), wg,
                     expert_W.astype(jnp.bfloat16))


# device time: 395696 ns/iter; 4.7373x vs baseline; 1.7915x over previous
import functools

import jax
import jax.numpy as jnp
from jax import lax
from jax.experimental import pallas as pl
from jax.experimental.pallas import tpu as pltpu

N_DEV = 8
E_LOC = 8
N_EXP = N_DEV * E_LOC
D = 512
D2 = D // 2
H = 1024


def _ring_moe(x, wg, w_top, w_bot):
    n = x.shape[0]

    def body(x_ref, wg_ref, wt_hbm, wb_hbm, out_ref, comm_r, comm_l,
             send_r, recv_r, send_l, recv_l, load_sem, credit_r, credit_l):
        s = pl.program_id(0)
        my = lax.axis_index("i")
        left = lax.rem(my + (N_DEV - 1), N_DEV)
        right = lax.rem(my + 1, N_DEV)
        cur = lax.rem(s, 2)
        nxt = lax.rem(s + 1, 2)

        @pl.when(s == 0)
        def _():
            barrier = pltpu.get_barrier_semaphore()
            for nbr in (left, right):
                pl.semaphore_signal(barrier, inc=1, device_id=(nbr,),
                                    device_id_type=pl.DeviceIdType.MESH)
            pl.semaphore_wait(barrier, 2)
            cp_t = pltpu.make_async_copy(wt_hbm, comm_r.at[0], load_sem)
            cp_t.start()
            cp_t.wait()
            cp_b = pltpu.make_async_copy(wb_hbm, comm_l.at[0], load_sem)
            cp_b.start()
            cp_b.wait()
            out_ref[:, :] = jnp.zeros_like(out_ref)

        rdma_r = pltpu.make_async_remote_copy(
            src_ref=comm_r.at[cur], dst_ref=comm_r.at[nxt],
            send_sem=send_r.at[cur], recv_sem=recv_r.at[nxt],
            device_id=(right,), device_id_type=pl.DeviceIdType.MESH)
        rdma_l = pltpu.make_async_remote_copy(
            src_ref=comm_l.at[cur], dst_ref=comm_l.at[nxt],
            send_sem=send_l.at[cur], recv_sem=recv_l.at[nxt],
            device_id=(left,), device_id_type=pl.DeviceIdType.MESH)

        @pl.when(s < N_DEV - 1)
        def _():
            @pl.when(s >= 1)
            def _():
                pl.semaphore_wait(credit_r, 1)
                pl.semaphore_wait(credit_l, 1)
            rdma_r.start()
            rdma_l.start()

        xt = x_ref[:, :D2]
        xb = x_ref[:, D2:]
        acc = None
        for j in range(E_LOC):
            yt = jnp.dot(xt, comm_r[cur, j],
                         preferred_element_type=jnp.float32)
            ct = wg_ref[0, :, j:j + 1]
            acc = ct * yt if acc is None else acc + ct * yt
            yb = jnp.dot(xb, comm_l[cur, j],
                         preferred_element_type=jnp.float32)
            cb = wg_ref[0, :, E_LOC + j:E_LOC + j + 1]
            acc = acc + cb * yb
        out_ref[:, :] = out_ref[:, :] + acc

        @pl.when(s < N_DEV - 1)
        def _():
            rdma_r.wait()
            rdma_l.wait()

            @pl.when(s < N_DEV - 2)
            def _():
                pl.semaphore_signal(credit_r, inc=1, device_id=(left,),
                                    device_id_type=pl.DeviceIdType.MESH)
                pl.semaphore_signal(credit_l, inc=1, device_id=(right,),
                                    device_id_type=pl.DeviceIdType.MESH)

        @pl.when(s == N_DEV - 1)
        def _():
            @functools.partial(pl.run_scoped,
                               exit_sem=pltpu.SemaphoreType.REGULAR)
            def _(exit_sem):
                for nbr in (left, right):
                    pl.semaphore_signal(exit_sem, inc=1, device_id=(nbr,),
                                        device_id_type=pl.DeviceIdType.MESH)
                pl.semaphore_wait(exit_sem, 2)

    return pl.pallas_call(
        body,
        grid=(N_DEV,),
        out_shape=jax.ShapeDtypeStruct((n, H), jnp.float32),
        in_specs=[
            pl.BlockSpec((n, D), lambda s: (0, 0)),
            pl.BlockSpec((1, n, 2 * E_LOC), lambda s: (s, 0, 0)),
            pl.BlockSpec(memory_space=pl.ANY),
            pl.BlockSpec(memory_space=pl.ANY),
        ],
        out_specs=pl.BlockSpec((n, H), lambda s: (0, 0)),
        scratch_shapes=[
            pltpu.VMEM((2, E_LOC, D2, H), jnp.bfloat16),
            pltpu.VMEM((2, E_LOC, D2, H), jnp.bfloat16),
            pltpu.SemaphoreType.DMA((2,)),
            pltpu.SemaphoreType.DMA((2,)),
            pltpu.SemaphoreType.DMA((2,)),
            pltpu.SemaphoreType.DMA((2,)),
            pltpu.SemaphoreType.DMA,
            pltpu.SemaphoreType.REGULAR,
            pltpu.SemaphoreType.REGULAR,
        ],
        compiler_params=pltpu.CompilerParams(
            collective_id=0,
            vmem_limit_bytes=60 * 1024 * 1024,
            dimension_semantics=("arbitrary",),
        ),
    )(x, wg, w_top, w_bot)


def kernel(x, router_W, route_idx, expert_W):
    n = x.shape[0]

    scores = x @ router_W
    probs = jax.nn.softmax(scores, axis=-1)

    eids = jnp.arange(N_EXP, dtype=route_idx.dtype)[None, :]
    oh0 = route_idx[:, 0:1] == eids
    oh1 = route_idx[:, 1:2] == eids
    g0 = jnp.sum(jnp.where(oh0, probs, 0.0), axis=-1, keepdims=True)
    g1 = jnp.sum(jnp.where(oh1, probs, 0.0), axis=-1, keepdims=True)
    gs = g0 + g1
    w_dense = jnp.where(oh0, g0 / gs, 0.0) + jnp.where(oh1, g1 / gs, 0.0)

    my = lax.axis_index("i")
    blocks = w_dense.reshape(n, N_DEV, E_LOC)
    wg_r = jnp.roll(blocks[:, ::-1, :], shift=my + 1, axis=1)
    wg_l = jnp.roll(blocks, shift=-my, axis=1)
    wg = jnp.concatenate([wg_r, wg_l], axis=2)
    wg = jnp.transpose(wg, (1, 0, 2))

    w_bf = expert_W.astype(jnp.bfloat16)
    return _ring_moe(x.astype(jnp.bfloat16), wg,
                     w_bf[:, :D2, :], w_bf[:, D2:, :])


# device time: 393924 ns/iter; 4.7586x vs baseline; 1.0045x over previous
import functools

import jax
import jax.numpy as jnp
from jax import lax
from jax.experimental import pallas as pl
from jax.experimental.pallas import tpu as pltpu

N_DEV = 8
E_LOC = 8
N_EXP = N_DEV * E_LOC
D = 512
D2 = D // 2
H = 1024


def _ring_moe(x, wg, w_top, w_bot):
    n = x.shape[0]

    def body(x_ref, wg_ref, wt_hbm, wb_hbm, out_ref, comm_r, comm_l,
             send_r, recv_r, send_l, recv_l, load_sem, credit_r, credit_l):
        s = pl.program_id(0)
        my = lax.axis_index("i")
        left = lax.rem(my + (N_DEV - 1), N_DEV)
        right = lax.rem(my + 1, N_DEV)
        cur = lax.rem(s, 2)
        nxt = lax.rem(s + 1, 2)

        @pl.when(s == 0)
        def _():
            barrier = pltpu.get_barrier_semaphore()
            for nbr in (left, right):
                pl.semaphore_signal(barrier, inc=1, device_id=(nbr,),
                                    device_id_type=pl.DeviceIdType.MESH)
            pl.semaphore_wait(barrier, 2)
            cp_t = pltpu.make_async_copy(wt_hbm, comm_r.at[0], load_sem)
            cp_t.start()
            cp_t.wait()
            cp_b = pltpu.make_async_copy(wb_hbm, comm_l.at[0], load_sem)
            cp_b.start()
            cp_b.wait()
            out_ref[:, :] = jnp.zeros_like(out_ref)

        rdma_r = pltpu.make_async_remote_copy(
            src_ref=comm_r.at[cur], dst_ref=comm_r.at[nxt],
            send_sem=send_r.at[cur], recv_sem=recv_r.at[nxt],
            device_id=(right,), device_id_type=pl.DeviceIdType.MESH)
        rdma_l = pltpu.make_async_remote_copy(
            src_ref=comm_l.at[cur], dst_ref=comm_l.at[nxt],
            send_sem=send_l.at[cur], recv_sem=recv_l.at[nxt],
            device_id=(left,), device_id_type=pl.DeviceIdType.MESH)

        @pl.when(s < N_DEV - 1)
        def _():
            @pl.when(s >= 1)
            def _():
                pl.semaphore_wait(credit_r, 1)
                pl.semaphore_wait(credit_l, 1)
            rdma_r.start()
            rdma_l.start()

        xt = x_ref[:, :D2]
        xb = x_ref[:, D2:]
        gxt = jnp.concatenate(
            [xt * wg_ref[0, :, j:j + 1].astype(jnp.bfloat16)
             for j in range(E_LOC)], axis=1)
        gxb = jnp.concatenate(
            [xb * wg_ref[0, :, E_LOC + j:E_LOC + j + 1].astype(jnp.bfloat16)
             for j in range(E_LOC)], axis=1)
        acc = jnp.dot(gxt, comm_r[cur].reshape(E_LOC * D2, H),
                      preferred_element_type=jnp.float32)
        acc = acc + jnp.dot(gxb, comm_l[cur].reshape(E_LOC * D2, H),
                            preferred_element_type=jnp.float32)
        out_ref[:, :] = out_ref[:, :] + acc

        @pl.when(s < N_DEV - 1)
        def _():
            rdma_r.wait()
            rdma_l.wait()

            @pl.when(s < N_DEV - 2)
            def _():
                pl.semaphore_signal(credit_r, inc=1, device_id=(left,),
                                    device_id_type=pl.DeviceIdType.MESH)
                pl.semaphore_signal(credit_l, inc=1, device_id=(right,),
                                    device_id_type=pl.DeviceIdType.MESH)

        @pl.when(s == N_DEV - 1)
        def _():
            @functools.partial(pl.run_scoped,
                               exit_sem=pltpu.SemaphoreType.REGULAR)
            def _(exit_sem):
                for nbr in (left, right):
                    pl.semaphore_signal(exit_sem, inc=1, device_id=(nbr,),
                                        device_id_type=pl.DeviceIdType.MESH)
                pl.semaphore_wait(exit_sem, 2)

    return pl.pallas_call(
        body,
        grid=(N_DEV,),
        out_shape=jax.ShapeDtypeStruct((n, H), jnp.float32),
        in_specs=[
            pl.BlockSpec((n, D), lambda s: (0, 0)),
            pl.BlockSpec((1, n, 2 * E_LOC), lambda s: (s, 0, 0)),
            pl.BlockSpec(memory_space=pl.ANY),
            pl.BlockSpec(memory_space=pl.ANY),
        ],
        out_specs=pl.BlockSpec((n, H), lambda s: (0, 0)),
        scratch_shapes=[
            pltpu.VMEM((2, E_LOC, D2, H), jnp.bfloat16),
            pltpu.VMEM((2, E_LOC, D2, H), jnp.bfloat16),
            pltpu.SemaphoreType.DMA((2,)),
            pltpu.SemaphoreType.DMA((2,)),
            pltpu.SemaphoreType.DMA((2,)),
            pltpu.SemaphoreType.DMA((2,)),
            pltpu.SemaphoreType.DMA,
            pltpu.SemaphoreType.REGULAR,
            pltpu.SemaphoreType.REGULAR,
        ],
        compiler_params=pltpu.CompilerParams(
            collective_id=0,
            vmem_limit_bytes=60 * 1024 * 1024,
            dimension_semantics=("arbitrary",),
        ),
    )(x, wg, w_top, w_bot)


def kernel(x, router_W, route_idx, expert_W):
    n = x.shape[0]

    scores = x @ router_W
    probs = jax.nn.softmax(scores, axis=-1)

    eids = jnp.arange(N_EXP, dtype=route_idx.dtype)[None, :]
    oh0 = route_idx[:, 0:1] == eids
    oh1 = route_idx[:, 1:2] == eids
    g0 = jnp.sum(jnp.where(oh0, probs, 0.0), axis=-1, keepdims=True)
    g1 = jnp.sum(jnp.where(oh1, probs, 0.0), axis=-1, keepdims=True)
    gs = g0 + g1
    w_dense = jnp.where(oh0, g0 / gs, 0.0) + jnp.where(oh1, g1 / gs, 0.0)

    my = lax.axis_index("i")
    blocks = w_dense.reshape(n, N_DEV, E_LOC)
    wg_r = jnp.roll(blocks[:, ::-1, :], shift=my + 1, axis=1)
    wg_l = jnp.roll(blocks, shift=-my, axis=1)
    wg = jnp.concatenate([wg_r, wg_l], axis=2)
    wg = jnp.transpose(wg, (1, 0, 2))

    w_bf = expert_W.astype(jnp.bfloat16)
    return _ring_moe(x.astype(jnp.bfloat16), wg,
                     w_bf[:, :D2, :], w_bf[:, D2:, :])


# device time: 365530 ns/iter; 5.1282x vs baseline; 1.0777x over previous
import functools

import jax
import jax.numpy as jnp
from jax import lax
from jax.experimental import pallas as pl
from jax.experimental.pallas import tpu as pltpu

N_DEV = 8
E_LOC = 8
N_EXP = N_DEV * E_LOC
D = 512
D2 = D // 2
H = 1024
N_STEP = 7


def _plane_moe(x, wg, wz, w_top, w_bot):
    n = x.shape[0]

    def body(x_ref, wg_ref, wz_ref, wt_hbm, wb_hbm, out_ref,
             comm_r, comm_l, z_top, z_bot,
             send_r, recv_r, send_l, recv_l,
             zsend, zrecv, load_sem, credit_r, credit_l):
        s = pl.program_id(0)
        my = lax.axis_index("i")
        p = lax.rem(my, 4)
        base = my - p
        right = base + lax.rem(p + 1, 4)
        left = base + lax.rem(p + 3, 4)
        zn = lax.rem(my + 4, N_DEV)
        cur = lax.rem(s, 2)
        nxt = lax.rem(s + 1, 2)

        zx_t = pltpu.make_async_remote_copy(
            src_ref=wt_hbm, dst_ref=z_top,
            send_sem=zsend.at[0], recv_sem=zrecv.at[0],
            device_id=(zn,), device_id_type=pl.DeviceIdType.MESH)
        zx_b = pltpu.make_async_remote_copy(
            src_ref=wb_hbm, dst_ref=z_bot,
            send_sem=zsend.at[1], recv_sem=zrecv.at[1],
            device_id=(zn,), device_id_type=pl.DeviceIdType.MESH)

        @pl.when(s == 0)
        def _():
            barrier = pltpu.get_barrier_semaphore()
            for nbr in (left, right, zn):
                pl.semaphore_signal(barrier, inc=1, device_id=(nbr,),
                                    device_id_type=pl.DeviceIdType.MESH)
            pl.semaphore_wait(barrier, 3)
            zx_t.start()
            zx_b.start()
            cp_t = pltpu.make_async_copy(wt_hbm, comm_r.at[0], load_sem)
            cp_t.start()
            cp_t.wait()
            cp_b = pltpu.make_async_copy(wb_hbm, comm_l.at[0], load_sem)
            cp_b.start()
            cp_b.wait()
            out_ref[:, :] = jnp.zeros_like(out_ref)

        @pl.when(s == 3)
        def _():
            zx_t.wait()
            zx_b.wait()

        rdma_r = pltpu.make_async_remote_copy(
            src_ref=comm_r.at[cur], dst_ref=comm_r.at[nxt],
            send_sem=send_r.at[cur], recv_sem=recv_r.at[nxt],
            device_id=(right,), device_id_type=pl.DeviceIdType.MESH)
        rdma_l = pltpu.make_async_remote_copy(
            src_ref=comm_l.at[cur], dst_ref=comm_l.at[nxt],
            send_sem=send_l.at[cur], recv_sem=recv_l.at[nxt],
            device_id=(left,), device_id_type=pl.DeviceIdType.MESH)
        zrdma_r = pltpu.make_async_remote_copy(
            src_ref=z_top, dst_ref=comm_r.at[nxt],
            send_sem=send_r.at[cur], recv_sem=recv_r.at[nxt],
            device_id=(right,), device_id_type=pl.DeviceIdType.MESH)
        zrdma_l = pltpu.make_async_remote_copy(
            src_ref=z_bot, dst_ref=comm_l.at[nxt],
            send_sem=send_l.at[cur], recv_sem=recv_l.at[nxt],
            device_id=(left,), device_id_type=pl.DeviceIdType.MESH)

        @pl.when(s < N_STEP - 1)
        def _():
            @pl.when(s >= 1)
            def _():
                pl.semaphore_wait(credit_r, 1)
                pl.semaphore_wait(credit_l, 1)

            @pl.when(s == 3)
            def _():
                zrdma_r.start()
                zrdma_l.start()

            @pl.when(s != 3)
            def _():
                rdma_r.start()
                rdma_l.start()

        xt = x_ref[:, :D2]
        xb = x_ref[:, D2:]

        def add_contrib(xh, w_ref, cols):
            for g in range(2):
                gx = jnp.concatenate(
                    [xh * cols[:, 4 * g + j:4 * g + j + 1]
                     .astype(jnp.bfloat16) for j in range(4)], axis=1)
                wm = w_ref[4 * g:4 * g + 4, :, :]
                out_ref[:, :] = out_ref[:, :] + jnp.dot(
                    gx, wm.reshape(4 * D2, H),
                    preferred_element_type=jnp.float32)

        add_contrib(xt, comm_r.at[cur], wg_ref[0, :, :E_LOC])
        add_contrib(xb, comm_l.at[cur], wg_ref[0, :, E_LOC:])

        @pl.when(s == 3)
        def _():
            add_contrib(xt, z_top, wz_ref[:, :])
            add_contrib(xb, z_bot, wz_ref[:, :])

        @pl.when(s < N_STEP - 1)
        def _():
            rdma_r.wait()
            rdma_l.wait()

            @pl.when(s < N_STEP - 2)
            def _():
                pl.semaphore_signal(credit_r, inc=1, device_id=(left,),
                                    device_id_type=pl.DeviceIdType.MESH)
                pl.semaphore_signal(credit_l, inc=1, device_id=(right,),
                                    device_id_type=pl.DeviceIdType.MESH)

        @pl.when(s == N_STEP - 1)
        def _():
            @functools.partial(pl.run_scoped,
                               exit_sem=pltpu.SemaphoreType.REGULAR)
            def _(exit_sem):
                for nbr in (left, right, zn):
                    pl.semaphore_signal(exit_sem, inc=1, device_id=(nbr,),
                                        device_id_type=pl.DeviceIdType.MESH)
                pl.semaphore_wait(exit_sem, 3)

    return pl.pallas_call(
        body,
        grid=(N_STEP,),
        out_shape=jax.ShapeDtypeStruct((n, H), jnp.float32),
        in_specs=[
            pl.BlockSpec((n, D), lambda s: (0, 0)),
            pl.BlockSpec((1, n, 2 * E_LOC), lambda s: (s, 0, 0)),
            pl.BlockSpec((n, E_LOC), lambda s: (0, 0)),
            pl.BlockSpec(memory_space=pl.ANY),
            pl.BlockSpec(memory_space=pl.ANY),
        ],
        out_specs=pl.BlockSpec((n, H), lambda s: (0, 0)),
        scratch_shapes=[
            pltpu.VMEM((2, E_LOC, D2, H), jnp.bfloat16),
            pltpu.VMEM((2, E_LOC, D2, H), jnp.bfloat16),
            pltpu.VMEM((E_LOC, D2, H), jnp.bfloat16),
            pltpu.VMEM((E_LOC, D2, H), jnp.bfloat16),
            pltpu.SemaphoreType.DMA((2,)),
            pltpu.SemaphoreType.DMA((2,)),
            pltpu.SemaphoreType.DMA((2,)),
            pltpu.SemaphoreType.DMA((2,)),
            pltpu.SemaphoreType.DMA((2,)),
            pltpu.SemaphoreType.DMA((2,)),
            pltpu.SemaphoreType.DMA,
            pltpu.SemaphoreType.REGULAR,
            pltpu.SemaphoreType.REGULAR,
        ],
        compiler_params=pltpu.CompilerParams(
            collective_id=0,
            vmem_limit_bytes=60 * 1024 * 1024,
            dimension_semantics=("arbitrary",),
        ),
    )(x, wg, wz, w_top, w_bot)


def kernel(x, router_W, route_idx, expert_W):
    n = x.shape[0]

    scores = x @ router_W
    probs = jax.nn.softmax(scores, axis=-1)

    eids = jnp.arange(N_EXP, dtype=route_idx.dtype)[None, :]
    oh0 = route_idx[:, 0:1] == eids
    oh1 = route_idx[:, 1:2] == eids
    g0 = jnp.sum(jnp.where(oh0, probs, 0.0), axis=-1, keepdims=True)
    g1 = jnp.sum(jnp.where(oh1, probs, 0.0), axis=-1, keepdims=True)
    gs = g0 + g1
    w_dense = jnp.where(oh0, g0 / gs, 0.0) + jnp.where(oh1, g1 / gs, 0.0)

    my = lax.axis_index("i")
    p = my % 4
    base = my - p
    s_arr = jnp.arange(N_STEP)
    hop = jnp.where(s_arr <= 3, s_arr, s_arr - 3)
    plane_off = jnp.where(s_arr <= 3, 0, 4)
    idx_r = (base + (p - hop) % 4 + plane_off) % N_DEV
    idx_l = (base + (p + hop) % 4 + plane_off) % N_DEV

    blocks = w_dense.reshape(n, N_DEV, E_LOC)
    wg_r = jnp.take(blocks, idx_r, axis=1)
    wg_l = jnp.take(blocks, idx_l, axis=1)
    wg = jnp.concatenate([wg_r, wg_l], axis=2)
    wg = jnp.transpose(wg, (1, 0, 2))
    wz = blocks[:, (my + 4) % N_DEV, :]

    w_bf = expert_W.astype(jnp.bfloat16)
    return _plane_moe(x.astype(jnp.bfloat16), wg, wz,
                      w_bf[:, :D2, :], w_bf[:, D2:, :])


# device time: 342890 ns/iter; 5.4668x vs baseline; 1.0660x over previous
import functools

import jax
import jax.numpy as jnp
from jax import lax
from jax.experimental import pallas as pl
from jax.experimental.pallas import tpu as pltpu

N_DEV = 8
E_LOC = 8
N_EXP = N_DEV * E_LOC
D = 512
D2 = D // 2
H = 1024
N_STEP = 7


def _plane_moe(x, wg, wz, w_top, w_bot):
    n = x.shape[0]

    def body(x_ref, wg_ref, wz_ref, wt_hbm, wb_hbm, out_ref,
             comm_r, comm_l, z_top, z_bot,
             send_r, recv_r, send_l, recv_l,
             zsend, zrecv, load_sem, credit_r, credit_l):
        s = pl.program_id(0)
        my = lax.axis_index("i")
        p = lax.rem(my, 4)
        base = my - p
        right = base + lax.rem(p + 1, 4)
        left = base + lax.rem(p + 3, 4)
        zn = lax.rem(my + 4, N_DEV)
        cur = lax.rem(s, 2)
        nxt = lax.rem(s + 1, 2)

        zx_t = pltpu.make_async_remote_copy(
            src_ref=wt_hbm, dst_ref=z_top,
            send_sem=zsend.at[0], recv_sem=zrecv.at[0],
            device_id=(zn,), device_id_type=pl.DeviceIdType.MESH)
        zx_b = pltpu.make_async_remote_copy(
            src_ref=wb_hbm, dst_ref=z_bot,
            send_sem=zsend.at[1], recv_sem=zrecv.at[1],
            device_id=(zn,), device_id_type=pl.DeviceIdType.MESH)

        @pl.when(s == 0)
        def _():
            barrier = pltpu.get_barrier_semaphore()
            for nbr in (left, right, zn):
                pl.semaphore_signal(barrier, inc=1, device_id=(nbr,),
                                    device_id_type=pl.DeviceIdType.MESH)
            pl.semaphore_wait(barrier, 3)
            zx_t.start()
            zx_b.start()
            cp_t = pltpu.make_async_copy(wt_hbm, comm_r.at[0], load_sem)
            cp_t.start()
            cp_t.wait()
            cp_b = pltpu.make_async_copy(wb_hbm, comm_l.at[0], load_sem)
            cp_b.start()
            cp_b.wait()
            out_ref[:, :] = jnp.zeros_like(out_ref)

        @pl.when(s == 3)
        def _():
            zx_t.wait()
            zx_b.wait()

        rdma_r = pltpu.make_async_remote_copy(
            src_ref=comm_r.at[cur], dst_ref=comm_r.at[nxt],
            send_sem=send_r.at[cur], recv_sem=recv_r.at[nxt],
            device_id=(right,), device_id_type=pl.DeviceIdType.MESH)
        rdma_l = pltpu.make_async_remote_copy(
            src_ref=comm_l.at[cur], dst_ref=comm_l.at[nxt],
            send_sem=send_l.at[cur], recv_sem=recv_l.at[nxt],
            device_id=(left,), device_id_type=pl.DeviceIdType.MESH)
        zrdma_r = pltpu.make_async_remote_copy(
            src_ref=z_top, dst_ref=comm_r.at[nxt],
            send_sem=send_r.at[cur], recv_sem=recv_r.at[nxt],
            device_id=(right,), device_id_type=pl.DeviceIdType.MESH)
        zrdma_l = pltpu.make_async_remote_copy(
            src_ref=z_bot, dst_ref=comm_l.at[nxt],
            send_sem=send_l.at[cur], recv_sem=recv_l.at[nxt],
            device_id=(left,), device_id_type=pl.DeviceIdType.MESH)

        @pl.when(s < N_STEP - 1)
        def _():
            @pl.when(s >= 1)
            def _():
                pl.semaphore_wait(credit_r, 1)
                pl.semaphore_wait(credit_l, 1)

            @pl.when(s == 3)
            def _():
                zrdma_r.start()
                zrdma_l.start()

            @pl.when(s != 3)
            def _():
                rdma_r.start()
                rdma_l.start()

        xt = x_ref[:, :D2]
        xb = x_ref[:, D2:]

        def add_dir(xh, w_ref, cols):
            gx = jnp.concatenate(
                [xh * cols[:, j:j + 1].astype(jnp.bfloat16)
                 for j in range(E_LOC)], axis=1)
            wm = w_ref[:, :, :]
            out_ref[:, :] = out_ref[:, :] + jnp.dot(
                gx, wm.reshape(E_LOC * D2, H),
                preferred_element_type=jnp.float32)

        add_dir(xt, comm_r.at[cur], wg_ref[0, :, :E_LOC])
        add_dir(xb, comm_l.at[cur], wg_ref[0, :, E_LOC:])

        @pl.when(s == 3)
        def _():
            add_dir(xt, z_top, wz_ref[:, :])
            add_dir(xb, z_bot, wz_ref[:, :])

        @pl.when(s < N_STEP - 1)
        def _():
            rdma_r.wait()
            rdma_l.wait()

            @pl.when(s < N_STEP - 2)
            def _():
                pl.semaphore_signal(credit_r, inc=1, device_id=(left,),
                                    device_id_type=pl.DeviceIdType.MESH)
                pl.semaphore_signal(credit_l, inc=1, device_id=(right,),
                                    device_id_type=pl.DeviceIdType.MESH)

        @pl.when(s == N_STEP - 1)
        def _():
            @functools.partial(pl.run_scoped,
                               exit_sem=pltpu.SemaphoreType.REGULAR)
            def _(exit_sem):
                for nbr in (left, right, zn):
                    pl.semaphore_signal(exit_sem, inc=1, device_id=(nbr,),
                                        device_id_type=pl.DeviceIdType.MESH)
                pl.semaphore_wait(exit_sem, 3)

    return pl.pallas_call(
        body,
        grid=(N_STEP,),
        out_shape=jax.ShapeDtypeStruct((n, H), jnp.float32),
        in_specs=[
            pl.BlockSpec((n, D), lambda s: (0, 0)),
            pl.BlockSpec((1, n, 2 * E_LOC), lambda s: (s, 0, 0)),
            pl.BlockSpec((n, E_LOC), lambda s: (0, 0)),
            pl.BlockSpec(memory_space=pl.ANY),
            pl.BlockSpec(memory_space=pl.ANY),
        ],
        out_specs=pl.BlockSpec((n, H), lambda s: (0, 0)),
        scratch_shapes=[
            pltpu.VMEM((2, E_LOC, D2, H), jnp.bfloat16),
            pltpu.VMEM((2, E_LOC, D2, H), jnp.bfloat16),
            pltpu.VMEM((E_LOC, D2, H), jnp.bfloat16),
            pltpu.VMEM((E_LOC, D2, H), jnp.bfloat16),
            pltpu.SemaphoreType.DMA((2,)),
            pltpu.SemaphoreType.DMA((2,)),
            pltpu.SemaphoreType.DMA((2,)),
            pltpu.SemaphoreType.DMA((2,)),
            pltpu.SemaphoreType.DMA((2,)),
            pltpu.SemaphoreType.DMA((2,)),
            pltpu.SemaphoreType.DMA,
            pltpu.SemaphoreType.REGULAR,
            pltpu.SemaphoreType.REGULAR,
        ],
        compiler_params=pltpu.CompilerParams(
            collective_id=0,
            vmem_limit_bytes=62 * 1024 * 1024,
            dimension_semantics=("arbitrary",),
        ),
    )(x, wg, wz, w_top, w_bot)


def kernel(x, router_W, route_idx, expert_W):
    n = x.shape[0]

    scores = x @ router_W
    probs = jax.nn.softmax(scores, axis=-1)

    eids = jnp.arange(N_EXP, dtype=route_idx.dtype)[None, :]
    oh0 = route_idx[:, 0:1] == eids
    oh1 = route_idx[:, 1:2] == eids
    g0 = jnp.sum(jnp.where(oh0, probs, 0.0), axis=-1, keepdims=True)
    g1 = jnp.sum(jnp.where(oh1, probs, 0.0), axis=-1, keepdims=True)
    gs = g0 + g1
    w_dense = jnp.where(oh0, g0 / gs, 0.0) + jnp.where(oh1, g1 / gs, 0.0)

    my = lax.axis_index("i")
    p = my % 4
    base = my - p
    s_arr = jnp.arange(N_STEP)
    hop = jnp.where(s_arr <= 3, s_arr, s_arr - 3)
    plane_off = jnp.where(s_arr <= 3, 0, 4)
    idx_r = (base + (p - hop) % 4 + plane_off) % N_DEV
    idx_l = (base + (p + hop) % 4 + plane_off) % N_DEV

    blocks = w_dense.reshape(n, N_DEV, E_LOC)
    wg_r = jnp.take(blocks, idx_r, axis=1)
    wg_l = jnp.take(blocks, idx_l, axis=1)
    wg = jnp.concatenate([wg_r, wg_l], axis=2)
    wg = jnp.transpose(wg, (1, 0, 2))
    wz = blocks[:, (my + 4) % N_DEV, :]

    w_bf = expert_W.astype(jnp.bfloat16)
    return _plane_moe(x.astype(jnp.bfloat16), wg, wz,
                      w_bf[:, :D2, :], w_bf[:, D2:, :])
